# baseline (speedup 1.0000x reference)
import functools

import jax
import jax.numpy as jnp
from jax import lax
from jax.experimental import pallas as pl
from jax.experimental.pallas import tpu as pltpu
from jax.experimental.pallas import tpu_sc as plsc

B, N, K = 4, 1024, 16
IN_DIM, FEAT, DIM, POS_H, ATTN_H, NCLS = 6, 64, 512, 512, 1024, 13
BN_ROWS = B * N
PPAD = 128


def _bn_relu_body(x_ref, w_ref, b_ref, g_ref, beta_ref, out_ref):
    h = jnp.dot(x_ref[...], w_ref[...], preferred_element_type=jnp.float32)
    h = h + b_ref[...]
    mu = jnp.mean(h, axis=0, keepdims=True)
    var = jnp.mean((h - mu) * (h - mu), axis=0, keepdims=True)
    hn = (h - mu) / jnp.sqrt(var + 1e-5) * g_ref[...] + beta_ref[...]
    out_ref[...] = jnp.maximum(hn, 0.0)


def _bn_relu(x, w, b, g, beta):
    d_in, d_out = w.shape
    return pl.pallas_call(
        _bn_relu_body,
        out_shape=jax.ShapeDtypeStruct((BN_ROWS, d_out), jnp.float32),
    )(x, w, b.reshape(1, -1), g.reshape(1, -1), beta.reshape(1, -1))


def _bf16_bits(x):
    b = lax.bitcast_convert_type(x, jnp.uint32)
    return (b + jnp.uint32(0x7FFF) + ((b >> 16) & jnp.uint32(1))) >> 16


def _pack_pair(lo, hi):
    u = (_bf16_bits(hi) << 16) | _bf16_bits(lo)
    return lax.bitcast_convert_type(u, jnp.int32)


def _unpack_pair(p):
    u = lax.bitcast_convert_type(p, jnp.uint32)
    lo = lax.bitcast_convert_type(u << 16, jnp.float32)
    hi = lax.bitcast_convert_type(u & jnp.uint32(0xFFFF0000), jnp.float32)
    return jnp.concatenate([lo, hi], axis=1)


def _qkv_body(h_ref, w_ref, q_ref, k_ref, v_ref):
    qkv = jnp.dot(h_ref[...], w_ref[...], preferred_element_type=jnp.float32)
    q_ref[...] = qkv[:, :DIM]
    kk = qkv[:, DIM:2 * DIM]
    vv = qkv[:, 2 * DIM:]
    k_ref[...] = _pack_pair(kk[:, :_DH], kk[:, _DH:])
    v_ref[...] = _pack_pair(vv[:, :_DH], vv[:, _DH:])


def _qkv(h, w):
    R = 512
    outp = jax.ShapeDtypeStruct((BN_ROWS, _DH), jnp.int32)
    return pl.pallas_call(
        _qkv_body,
        grid=(BN_ROWS // R,),
        in_specs=[
            pl.BlockSpec((R, FEAT), lambda i: (i, 0)),
            pl.BlockSpec((FEAT, 3 * DIM), lambda i: (0, 0)),
        ],
        out_specs=[pl.BlockSpec((R, DIM), lambda i: (i, 0)),
                   pl.BlockSpec((R, _DH), lambda i: (i, 0)),
                   pl.BlockSpec((R, _DH), lambda i: (i, 0))],
        out_shape=[jax.ShapeDtypeStruct((BN_ROWS, DIM), jnp.float32), outp, outp],
    )(h, w)


def _topk_body(pos_ref, posT_ref, out_ref):
    b = pl.program_id(0)
    d2 = None
    for c in range(3):
        pj = pos_ref[0][:, c:c + 1]
        pi = posT_ref[0][c:c + 1, :]
        diff = pj - pi
        sq = diff * diff
        d2 = sq if d2 is None else d2 + sq
    work = d2
    jio = lax.broadcasted_iota(jnp.int32, work.shape, 0)
    base = b * N
    for t in range(K):
        m = jnp.min(work, axis=0, keepdims=True)
        cand = jnp.where(work == m, jio, N)
        jmin = jnp.min(cand, axis=0, keepdims=True)
        out_ref[0, t:t + 1, :] = jmin + base
        work = jnp.where(jio == jmin, jnp.inf, work)


def _topk(pos, posT):
    R = 256
    return pl.pallas_call(
        _topk_body,
        grid=(B, N // R),
        in_specs=[
            pl.BlockSpec((1, N, 3), lambda b, i: (b, 0, 0)),
            pl.BlockSpec((1, 3, R), lambda b, i: (b, 0, i)),
        ],
        out_specs=pl.BlockSpec((1, K, R), lambda b, i: (b, 0, i)),
        out_shape=jax.ShapeDtypeStruct((B, K, N), jnp.int32),
    )(pos, posT)


_NCHUNKS = (512, 512)
_NW = 32
_CH = 128
_DH = DIM // 2


def _gather_kv_kernel(nrows, kt_hbm, vt_hbm, idx_hbm, kout, vout,
                      idx_v, krows, vrows, sem):
    wid = lax.axis_index("s") * 2 + lax.axis_index("c")
    rpw = nrows // _NW
    base = wid * rpw

    def body(ci, _):
        off = pl.multiple_of(base + ci * _CH, 8)
        pltpu.sync_copy(idx_hbm.at[pl.ds(off, _CH)], idx_v)
        pltpu.async_copy(kt_hbm.at[idx_v], krows, sem).wait()
        pltpu.sync_copy(krows, kout.at[pl.ds(off, _CH)])
        pltpu.async_copy(vt_hbm.at[idx_v], vrows, sem).wait()
        pltpu.sync_copy(vrows, vout.at[pl.ds(off, _CH)])
        return _

    lax.fori_loop(0, rpw // _CH, body, 0)


def _gather_kv(k_table, v_table, idx_chunk):
    nrows = idx_chunk.shape[0]
    fn = functools.partial(
        pl.kernel,
        mesh=plsc.VectorSubcoreMesh(core_axis_name="c", subcore_axis_name="s"),
        out_type=[
            jax.ShapeDtypeStruct((nrows, _DH), jnp.int32),
            jax.ShapeDtypeStruct((nrows, _DH), jnp.int32),
        ],
        scratch_types=[
            pltpu.VMEM((_CH,), jnp.int32),
            pltpu.VMEM((_CH, _DH), jnp.int32),
            pltpu.VMEM((_CH, _DH), jnp.int32),
            pltpu.SemaphoreType.DMA,
        ],
    )(functools.partial(_gather_kv_kernel, nrows))
    return fn(k_table, v_table, idx_chunk)


def _gather_pos_kernel(nrows, pt_hbm, idx_hbm, pout, idx_v, prows, sem):
    wid = lax.axis_index("s") * 2 + lax.axis_index("c")
    rpw = nrows // _NW
    base = wid * rpw

    def body(ci, _):
        off = pl.multiple_of(base + ci * _CH, 8)
        pltpu.sync_copy(idx_hbm.at[pl.ds(off, _CH)], idx_v)
        pltpu.async_copy(pt_hbm.at[idx_v], prows, sem).wait()
        pltpu.sync_copy(prows, pout.at[pl.ds(off, _CH)])
        return _

    lax.fori_loop(0, rpw // _CH, body, 0)


def _gather_pos(p_table, idx_chunk):
    nrows = idx_chunk.shape[0]
    fn = functools.partial(
        pl.kernel,
        mesh=plsc.VectorSubcoreMesh(core_axis_name="c", subcore_axis_name="s"),
        out_type=jax.ShapeDtypeStruct((nrows, PPAD), jnp.float32),
        scratch_types=[
            pltpu.VMEM((_CH,), jnp.int32),
            pltpu.VMEM((_CH, PPAD), jnp.float32),
            pltpu.SemaphoreType.DMA,
        ],
    )(functools.partial(_gather_pos_kernel, nrows))
    return fn(p_table, idx_chunk)


def _attn_body(q_ref, x_ref, pp_ref, kn_ref, vn_ref, pn_ref,
               wp1_ref, bp1_ref, wp2_ref, bp2_ref,
               wa1_ref, ba1_ref, wa2_ref, ba2_ref,
               wo_ref, bo_ref, out_ref, m_ref, s_ref, a_ref):
    k = pl.program_id(2)
    q = q_ref[0]
    pp = pp_ref[0]
    kn = _unpack_pair(kn_ref[0, 0])
    vn = _unpack_pair(vn_ref[0, 0])
    pn = pn_ref[0, 0]
    rel = (pp - pn).astype(jnp.bfloat16)
    e1 = jnp.maximum(
        jnp.dot(rel, wp1_ref[...], preferred_element_type=jnp.float32)
        + bp1_ref[...], 0.0)
    remb = jnp.dot(e1.astype(jnp.bfloat16), wp2_ref[...],
                   preferred_element_type=jnp.float32) + bp2_ref[...]
    t = (q - kn + remb).astype(jnp.bfloat16)
    h = jnp.maximum(
        jnp.dot(t, wa1_ref[...], preferred_element_type=jnp.float32)
        + ba1_ref[...], 0.0)
    sim = jnp.dot(h.astype(jnp.bfloat16), wa2_ref[...],
                  preferred_element_type=jnp.float32) + ba2_ref[...]
    u = vn + remb

    @pl.when(k == 0)
    def _init():
        m_ref[...] = sim
        s_ref[...] = jnp.ones_like(sim)
        a_ref[...] = u

    @pl.when(k > 0)
    def _update():
        m_old = m_ref[...]
        m_new = jnp.maximum(m_old, sim)
        alpha = jnp.exp(m_old - m_new)
        p = jnp.exp(sim - m_new)
        m_ref[...] = m_new
        s_ref[...] = s_ref[...] * alpha + p
        a_ref[...] = a_ref[...] * alpha + p * u

    @pl.when(k == K - 1)
    def _final():
        agg = (a_ref[...] / s_ref[...]).astype(jnp.bfloat16)
        out_ref[0] = x_ref[0] + (
            jnp.dot(agg, wo_ref[...], preferred_element_type=jnp.float32)
            + bo_ref[...])


def _attn(q, x_res, pos_pad, kn, vn, pn, wp1p, lp, noff, nh):
    P = 512 if (nh % 512 == 0 and noff % 512 == 0) else 256
    bf = jnp.bfloat16
    off = noff // P
    wspec = lambda a: pl.BlockSpec(a.shape, lambda b, i, k: (0,) * a.ndim)
    args = [
        q.reshape(B, N, DIM), x_res.reshape(B, N, FEAT),
        pos_pad.reshape(B, N, PPAD),
        kn.reshape(B, K, nh, _DH), vn.reshape(B, K, nh, _DH),
        pn.reshape(B, K, nh, PPAD),
        wp1p, lp['bp1'].reshape(1, -1),
        lp['Wp2'].astype(bf), lp['bp2'].reshape(1, -1),
        lp['Wa1'].astype(bf), lp['ba1'].reshape(1, -1),
        lp['Wa2'].astype(bf), lp['ba2'].reshape(1, -1),
        lp['Wo'].astype(bf), lp['bo'].reshape(1, -1),
    ]
    in_specs = [
        pl.BlockSpec((1, P, DIM), lambda b, i, k: (b, i + off, 0)),
        pl.BlockSpec((1, P, FEAT), lambda b, i, k: (b, i + off, 0)),
        pl.BlockSpec((1, P, PPAD), lambda b, i, k: (b, i + off, 0)),
        pl.BlockSpec((1, 1, P, _DH), lambda b, i, k: (b, k, i, 0)),
        pl.BlockSpec((1, 1, P, _DH), lambda b, i, k: (b, k, i, 0)),
        pl.BlockSpec((1, 1, P, PPAD), lambda b, i, k: (b, k, i, 0)),
    ] + [wspec(a) for a in args[6:]]
    return pl.pallas_call(
        _attn_body,
        grid=(B, nh // P, K),
        in_specs=in_specs,
        out_specs=pl.BlockSpec((1, P, FEAT), lambda b, i, k: (b, i, 0)),
        out_shape=jax.ShapeDtypeStruct((B, nh, FEAT), jnp.float32),
        scratch_shapes=[pltpu.VMEM((P, DIM), jnp.float32)] * 3,
    )(*args)


def _head_body(h_ref, w1_ref, b1_ref, g1_ref, be1_ref,
               w2_ref, b2_ref, g2_ref, be2_ref, w3_ref, b3_ref, out_ref):
    def bn_relu(h, g, beta):
        mu = jnp.mean(h, axis=0, keepdims=True)
        var = jnp.mean((h - mu) * (h - mu), axis=0, keepdims=True)
        return jnp.maximum((h - mu) / jnp.sqrt(var + 1e-5) * g + beta, 0.0)

    h1 = bn_relu(
        jnp.dot(h_ref[...], w1_ref[...], preferred_element_type=jnp.float32)
        + b1_ref[...], g1_ref[...], be1_ref[...])
    h2 = bn_relu(
        jnp.dot(h1, w2_ref[...], preferred_element_type=jnp.float32)
        + b2_ref[...], g2_ref[...], be2_ref[...])
    out_ref[...] = (
        jnp.dot(h2, w3_ref[...], preferred_element_type=jnp.float32)
        + b3_ref[...])


def _head(h, p):
    return pl.pallas_call(
        _head_body,
        out_shape=jax.ShapeDtypeStruct((BN_ROWS, NCLS), jnp.float32),
    )(h, p['wc1'], p['bc1'].reshape(1, -1), p['bnc1_g'].reshape(1, -1),
      p['bnc1_b'].reshape(1, -1), p['wc2'], p['bc2'].reshape(1, -1),
      p['bnc2_g'].reshape(1, -1), p['bnc2_b'].reshape(1, -1),
      p['wc3'], p['bc3'].reshape(1, -1))


def kernel(x, pos, params):
    p = params
    xf = x.reshape(BN_ROWS, IN_DIM)
    posf = pos.reshape(BN_ROWS, 3)
    pos_pad = jnp.concatenate(
        [posf, jnp.zeros((BN_ROWS, PPAD - 3), jnp.float32)], axis=1)
    posT = pos.transpose(0, 2, 1)

    gidx = _topk(pos, posT)
    bounds = []
    n0 = 0
    for nh in _NCHUNKS:
        bounds.append((n0, nh))
        n0 += nh
    idxs = [gidx[:, :, n0:n0 + nh].reshape(-1) for n0, nh in bounds]
    pns = [_gather_pos(pos_pad, ix) for ix in idxs]

    h = _bn_relu(xf, p['w0'], p['b0'], p['bn0_g'], p['bn0_b'])
    for lname, mid in ((('ptl_down'), True), (('ptl_up'), False)):
        lp = p[lname]
        wp1p = jnp.zeros((PPAD, POS_H), jnp.bfloat16).at[:3].set(
            lp['Wp1'].astype(jnp.bfloat16))
        q, kf, vf = _qkv(h, lp['Wqkv'])
        kvs = [_gather_kv(kf, vf, ix) for ix in idxs]
        parts = [
            _attn(q, h, pos_pad, kv[0], kv[1], pn, wp1p, lp, n0, nh)
            for kv, pn, (n0, nh) in zip(kvs, pns, bounds)
        ]
        h = jnp.concatenate(parts, axis=1).reshape(BN_ROWS, FEAT)
        if mid:
            h = _bn_relu(h, p['w1'], p['b1'], p['bn1_g'], p['bn1_b'])

    logits = _head(h, p)
    return logits.reshape(B, N, NCLS)


# speedup vs baseline: 1.0691x; 1.0691x over previous
import functools

import jax
import jax.numpy as jnp
from jax import lax
from jax.experimental import pallas as pl
from jax.experimental.pallas import tpu as pltpu
from jax.experimental.pallas import tpu_sc as plsc

B, N, K = 4, 1024, 16
IN_DIM, FEAT, DIM, POS_H, ATTN_H, NCLS = 6, 64, 512, 512, 1024, 13
BN_ROWS = B * N
PPAD = 128


def _bn_relu_body(x_ref, w_ref, b_ref, g_ref, beta_ref, out_ref):
    h = jnp.dot(x_ref[...], w_ref[...], preferred_element_type=jnp.float32)
    h = h + b_ref[...]
    mu = jnp.mean(h, axis=0, keepdims=True)
    var = jnp.mean((h - mu) * (h - mu), axis=0, keepdims=True)
    hn = (h - mu) / jnp.sqrt(var + 1e-5) * g_ref[...] + beta_ref[...]
    out_ref[...] = jnp.maximum(hn, 0.0)


def _bn_relu(x, w, b, g, beta):
    d_in, d_out = w.shape
    return pl.pallas_call(
        _bn_relu_body,
        out_shape=jax.ShapeDtypeStruct((BN_ROWS, d_out), jnp.float32),
    )(x, w, b.reshape(1, -1), g.reshape(1, -1), beta.reshape(1, -1))


def _bf16_bits(x):
    b = lax.bitcast_convert_type(x, jnp.uint32)
    return (b + jnp.uint32(0x7FFF) + ((b >> 16) & jnp.uint32(1))) >> 16


def _pack_pair(lo, hi):
    u = (_bf16_bits(hi) << 16) | _bf16_bits(lo)
    return lax.bitcast_convert_type(u, jnp.int32)


def _unpack_pair(p):
    u = lax.bitcast_convert_type(p, jnp.uint32)
    lo = lax.bitcast_convert_type(u << 16, jnp.float32)
    hi = lax.bitcast_convert_type(u & jnp.uint32(0xFFFF0000), jnp.float32)
    return jnp.concatenate([lo, hi], axis=1)


def _qkv_body(h_ref, w_ref, q_ref, kv_ref):
    qkv = jnp.dot(h_ref[...], w_ref[...], preferred_element_type=jnp.float32)
    q_ref[...] = qkv[:, :DIM]
    kk = qkv[:, DIM:2 * DIM]
    vv = qkv[:, 2 * DIM:]
    kv_ref[...] = jnp.concatenate(
        [_pack_pair(kk[:, :_DH], kk[:, _DH:]),
         _pack_pair(vv[:, :_DH], vv[:, _DH:])], axis=1)


def _qkv(h, w):
    R = 512
    return pl.pallas_call(
        _qkv_body,
        grid=(BN_ROWS // R,),
        in_specs=[
            pl.BlockSpec((R, FEAT), lambda i: (i, 0)),
            pl.BlockSpec((FEAT, 3 * DIM), lambda i: (0, 0)),
        ],
        out_specs=[pl.BlockSpec((R, DIM), lambda i: (i, 0)),
                   pl.BlockSpec((R, 2 * _DH), lambda i: (i, 0))],
        out_shape=[jax.ShapeDtypeStruct((BN_ROWS, DIM), jnp.float32),
                   jax.ShapeDtypeStruct((BN_ROWS, 2 * _DH), jnp.int32)],
    )(h, w)


def _topk_body(pos_ref, posT_ref, out_ref):
    b = pl.program_id(0)
    d2 = None
    for c in range(3):
        pj = pos_ref[0][:, c:c + 1]
        pi = posT_ref[0][c:c + 1, :]
        diff = pj - pi
        sq = diff * diff
        d2 = sq if d2 is None else d2 + sq
    work = d2
    jio = lax.broadcasted_iota(jnp.int32, work.shape, 0)
    base = b * N
    for t in range(K):
        m = jnp.min(work, axis=0, keepdims=True)
        cand = jnp.where(work == m, jio, N)
        jmin = jnp.min(cand, axis=0, keepdims=True)
        out_ref[0, t:t + 1, :] = jmin + base
        work = jnp.where(jio == jmin, jnp.inf, work)


def _topk(pos, posT):
    R = 256
    return pl.pallas_call(
        _topk_body,
        grid=(B, N // R),
        in_specs=[
            pl.BlockSpec((1, N, 3), lambda b, i: (b, 0, 0)),
            pl.BlockSpec((1, 3, R), lambda b, i: (b, 0, i)),
        ],
        out_specs=pl.BlockSpec((1, K, R), lambda b, i: (b, 0, i)),
        out_shape=jax.ShapeDtypeStruct((B, K, N), jnp.int32),
    )(pos, posT)


_NCHUNKS = (512, 512)
_NW = 32
_CH = 128
_DH = DIM // 2


def _gather_kv_kernel(nrows, kvt_hbm, idx_hbm, kvout, idx_v, kvrows, sem):
    wid = lax.axis_index("s") * 2 + lax.axis_index("c")
    rpw = nrows // _NW
    base = wid * rpw

    def body(ci, _):
        off = pl.multiple_of(base + ci * _CH, 8)
        pltpu.sync_copy(idx_hbm.at[pl.ds(off, _CH)], idx_v)
        pltpu.async_copy(kvt_hbm.at[idx_v], kvrows, sem).wait()
        pltpu.sync_copy(kvrows, kvout.at[pl.ds(off, _CH)])
        return _

    lax.fori_loop(0, rpw // _CH, body, 0)


def _gather_kv(kv_table, idx_chunk):
    nrows = idx_chunk.shape[0]
    fn = functools.partial(
        pl.kernel,
        mesh=plsc.VectorSubcoreMesh(core_axis_name="c", subcore_axis_name="s"),
        out_type=jax.ShapeDtypeStruct((nrows, 2 * _DH), jnp.int32),
        scratch_types=[
            pltpu.VMEM((_CH,), jnp.int32),
            pltpu.VMEM((_CH, 2 * _DH), jnp.int32),
            pltpu.SemaphoreType.DMA,
        ],
    )(functools.partial(_gather_kv_kernel, nrows))
    return fn(kv_table, idx_chunk)


def _gather_pos_kernel(nrows, pt_hbm, idx_hbm, pout, idx_v, prows, sem):
    wid = lax.axis_index("s") * 2 + lax.axis_index("c")
    rpw = nrows // _NW
    base = wid * rpw

    def body(ci, _):
        off = pl.multiple_of(base + ci * _CH, 8)
        pltpu.sync_copy(idx_hbm.at[pl.ds(off, _CH)], idx_v)
        pltpu.async_copy(pt_hbm.at[idx_v], prows, sem).wait()
        pltpu.sync_copy(prows, pout.at[pl.ds(off, _CH)])
        return _

    lax.fori_loop(0, rpw // _CH, body, 0)


def _gather_pos(p_table, idx_chunk):
    nrows = idx_chunk.shape[0]
    fn = functools.partial(
        pl.kernel,
        mesh=plsc.VectorSubcoreMesh(core_axis_name="c", subcore_axis_name="s"),
        out_type=jax.ShapeDtypeStruct((nrows, PPAD), jnp.float32),
        scratch_types=[
            pltpu.VMEM((_CH,), jnp.int32),
            pltpu.VMEM((_CH, PPAD), jnp.float32),
            pltpu.SemaphoreType.DMA,
        ],
    )(functools.partial(_gather_pos_kernel, nrows))
    return fn(p_table, idx_chunk)


def _attn_body(q_ref, x_ref, pp_ref, kv_ref, pn_ref,
               wp1_ref, bp1_ref, wp2_ref, bp2_ref,
               wa1_ref, ba1_ref, wa2_ref, ba2_ref,
               wo_ref, bo_ref, out_ref, m_ref, s_ref, a_ref):
    q = q_ref[0]
    pp = pp_ref[0]
    wp1 = wp1_ref[...]
    wp2 = wp2_ref[...]
    wa1 = wa1_ref[...]
    wa2 = wa2_ref[...]
    bp1 = bp1_ref[...]
    bp2 = bp2_ref[...]
    ba1 = ba1_ref[...]
    ba2 = ba2_ref[...]
    for k in range(K):
        kn = _unpack_pair(kv_ref[0, k, :, :_DH])
        vn = _unpack_pair(kv_ref[0, k, :, _DH:])
        pn = pn_ref[0, k]
        rel = (pp - pn).astype(jnp.bfloat16)
        e1 = jnp.maximum(
            jnp.dot(rel, wp1, preferred_element_type=jnp.float32) + bp1, 0.0)
        remb = jnp.dot(e1.astype(jnp.bfloat16), wp2,
                       preferred_element_type=jnp.float32) + bp2
        t = (q - kn + remb).astype(jnp.bfloat16)
        h = jnp.maximum(
            jnp.dot(t, wa1, preferred_element_type=jnp.float32) + ba1, 0.0)
        sim = jnp.dot(h.astype(jnp.bfloat16), wa2,
                      preferred_element_type=jnp.float32) + ba2
        u = vn + remb
        if k == 0:
            m_ref[...] = sim
            s_ref[...] = jnp.ones_like(sim)
            a_ref[...] = u
        else:
            m_old = m_ref[...]
            m_new = jnp.maximum(m_old, sim)
            alpha = jnp.exp(m_old - m_new)
            p = jnp.exp(sim - m_new)
            m_ref[...] = m_new
            s_ref[...] = s_ref[...] * alpha + p
            a_ref[...] = a_ref[...] * alpha + p * u
    agg = (a_ref[...] / s_ref[...]).astype(jnp.bfloat16)
    out_ref[0] = x_ref[0] + (
        jnp.dot(agg, wo_ref[...], preferred_element_type=jnp.float32)
        + bo_ref[...])


def _attn(q, x_res, pos_pad, kv, pn, wp1p, lp, noff, nh):
    P = 512 if (nh % 512 == 0 and noff % 512 == 0) else 256
    bf = jnp.bfloat16
    off = noff // P
    wspec = lambda a: pl.BlockSpec(a.shape, lambda b, i: (0,) * a.ndim)
    args = [
        q.reshape(B, N, DIM), x_res.reshape(B, N, FEAT),
        pos_pad.reshape(B, N, PPAD),
        kv.reshape(B, K, nh, 2 * _DH),
        pn.reshape(B, K, nh, PPAD),
        wp1p, lp['bp1'].reshape(1, -1),
        lp['Wp2'].astype(bf), lp['bp2'].reshape(1, -1),
        lp['Wa1'].astype(bf), lp['ba1'].reshape(1, -1),
        lp['Wa2'].astype(bf), lp['ba2'].reshape(1, -1),
        lp['Wo'].astype(bf), lp['bo'].reshape(1, -1),
    ]
    in_specs = [
        pl.BlockSpec((1, P, DIM), lambda b, i: (b, i + off, 0)),
        pl.BlockSpec((1, P, FEAT), lambda b, i: (b, i + off, 0)),
        pl.BlockSpec((1, P, PPAD), lambda b, i: (b, i + off, 0)),
        pl.BlockSpec((1, K, P, 2 * _DH), lambda b, i: (b, 0, i, 0)),
        pl.BlockSpec((1, K, P, PPAD), lambda b, i: (b, 0, i, 0)),
    ] + [wspec(a) for a in args[5:]]
    return pl.pallas_call(
        _attn_body,
        grid=(B, nh // P),
        in_specs=in_specs,
        out_specs=pl.BlockSpec((1, P, FEAT), lambda b, i: (b, i, 0)),
        out_shape=jax.ShapeDtypeStruct((B, nh, FEAT), jnp.float32),
        scratch_shapes=[pltpu.VMEM((P, DIM), jnp.float32)] * 3,
    )(*args)


def _head_body(h_ref, w1_ref, b1_ref, g1_ref, be1_ref,
               w2_ref, b2_ref, g2_ref, be2_ref, w3_ref, b3_ref, out_ref):
    def bn_relu(h, g, beta):
        mu = jnp.mean(h, axis=0, keepdims=True)
        var = jnp.mean((h - mu) * (h - mu), axis=0, keepdims=True)
        return jnp.maximum((h - mu) / jnp.sqrt(var + 1e-5) * g + beta, 0.0)

    h1 = bn_relu(
        jnp.dot(h_ref[...], w1_ref[...], preferred_element_type=jnp.float32)
        + b1_ref[...], g1_ref[...], be1_ref[...])
    h2 = bn_relu(
        jnp.dot(h1, w2_ref[...], preferred_element_type=jnp.float32)
        + b2_ref[...], g2_ref[...], be2_ref[...])
    out_ref[...] = (
        jnp.dot(h2, w3_ref[...], preferred_element_type=jnp.float32)
        + b3_ref[...])


def _head(h, p):
    return pl.pallas_call(
        _head_body,
        out_shape=jax.ShapeDtypeStruct((BN_ROWS, NCLS), jnp.float32),
    )(h, p['wc1'], p['bc1'].reshape(1, -1), p['bnc1_g'].reshape(1, -1),
      p['bnc1_b'].reshape(1, -1), p['wc2'], p['bc2'].reshape(1, -1),
      p['bnc2_g'].reshape(1, -1), p['bnc2_b'].reshape(1, -1),
      p['wc3'], p['bc3'].reshape(1, -1))


def kernel(x, pos, params):
    p = params
    xf = x.reshape(BN_ROWS, IN_DIM)
    posf = pos.reshape(BN_ROWS, 3)
    pos_pad = jnp.concatenate(
        [posf, jnp.zeros((BN_ROWS, PPAD - 3), jnp.float32)], axis=1)
    posT = pos.transpose(0, 2, 1)

    gidx = _topk(pos, posT)
    bounds = []
    n0 = 0
    for nh in _NCHUNKS:
        bounds.append((n0, nh))
        n0 += nh
    idxs = [gidx[:, :, n0:n0 + nh].reshape(-1) for n0, nh in bounds]
    pns = [_gather_pos(pos_pad, ix) for ix in idxs]

    h = _bn_relu(xf, p['w0'], p['b0'], p['bn0_g'], p['bn0_b'])
    for lname, mid in ((('ptl_down'), True), (('ptl_up'), False)):
        lp = p[lname]
        wp1p = jnp.zeros((PPAD, POS_H), jnp.bfloat16).at[:3].set(
            lp['Wp1'].astype(jnp.bfloat16))
        q, kvf = _qkv(h, lp['Wqkv'])
        kvs = [_gather_kv(kvf, ix) for ix in idxs]
        parts = [
            _attn(q, h, pos_pad, kv, pn, wp1p, lp, n0, nh)
            for kv, pn, (n0, nh) in zip(kvs, pns, bounds)
        ]
        h = jnp.concatenate(parts, axis=1).reshape(BN_ROWS, FEAT)
        if mid:
            h = _bn_relu(h, p['w1'], p['b1'], p['bn1_g'], p['bn1_b'])

    logits = _head(h, p)
    return logits.reshape(B, N, NCLS)
